# baseline (device time: 28380 ns/iter reference)
import jax
import jax.numpy as jnp
from jax import lax
from jax.experimental import pallas as pl
from jax.experimental.pallas import tpu as pltpu

_GROUPS = [(0, 256), (256, 256), (512, 128), (640, 128), (768, 128), (896, 128)]
_NG = len(_GROUPS)
_COMM = True


def kernel(x, w_mat):
    m, _ = x.shape
    _, n = w_mat.shape
    h2, h4 = m // 2, m // 4

    def body(x_ref, w_ref, out_ref, acc_ref, *rest):
        recvs = [rest[3 * g: 3 * g + 3] for g in range(_NG)]
        send_sems, recv_sems = rest[3 * _NG], rest[3 * _NG + 1]

        i = lax.axis_index("i")
        q = i & 3
        ycrd = q >> 1
        xcrd = (q & 1) ^ ycrd
        zcrd = i >> 2
        py, px, pz = i ^ 3, i ^ 1, i ^ 4

        orders3 = [
            ((ycrd, xcrd, zcrd), (py, px, pz)),
            ((xcrd, zcrd, ycrd), (px, pz, py)),
            ((zcrd, ycrd, xcrd), (pz, py, px)),
        ]
        crd = [orders3[g % 3][0] for g in range(_NG)]
        par = [orders3[g % 3][1] for g in range(_NG)]
        cols = [pl.ds(off, w) for off, w in _GROUPS]

        k0 = [crd[g][0] * h2 for g in range(_NG)]
        k1 = [k0[g] + crd[g][1] * h4 for g in range(_NG)]
        s0 = [(1 - crd[g][0]) * h2 for g in range(_NG)]
        s1 = [k0[g] + (1 - crd[g][1]) * h4 for g in range(_NG)]

        if _COMM:
            barrier_sem = pltpu.get_barrier_semaphore()
            for nbr in (py, px, pz):
                pl.semaphore_signal(
                    barrier_sem, inc=1,
                    device_id=(nbr,), device_id_type=pl.DeviceIdType.MESH,
                )

        def mm(rows, g):
            return jnp.dot(
                x_ref[rows, :], w_ref[:, cols[g]],
                preferred_element_type=jnp.float32,
            ).astype(jnp.bfloat16)

        class _Noop:
            def wait(self):
                pass

        def start(r, g, partner, src, dst):
            if not _COMM:
                return _Noop()
            rdma = pltpu.make_async_remote_copy(
                src_ref=src, dst_ref=dst,
                send_sem=send_sems.at[6 * g + r], recv_sem=recv_sems.at[6 * g + r],
                device_id=(partner,), device_id_type=pl.DeviceIdType.MESH,
            )
            rdma.start()
            return rdma

        acc_ref[pl.ds(s0[0], h2), cols[0]] = mm(pl.ds(s0[0], h2), 0)
        if _COMM:
            pl.semaphore_wait(barrier_sem, 3)
        rd = []
        for g in range(_NG):
            if g > 0:
                acc_ref[pl.ds(s0[g], h2), cols[g]] = mm(pl.ds(s0[g], h2), g)
            rd.append(start(0, g, par[g][0],
                            acc_ref.at[pl.ds(s0[g], h2), cols[g]], recvs[g][0]))
        for g in range(_NG):
            acc_ref[pl.ds(k0[g], h2), cols[g]] = mm(pl.ds(k0[g], h2), g)

        rd1 = []
        for g in range(_NG):
            rd[g].wait()
            o_s = s1[g] - k0[g]
            acc_ref[pl.ds(s1[g], h4), cols[g]] = (
                acc_ref[pl.ds(s1[g], h4), cols[g]]
                + recvs[g][0][pl.ds(o_s, h4), :]
            )
            rd1.append(start(1, g, par[g][1],
                             acc_ref.at[pl.ds(s1[g], h4), cols[g]], recvs[g][1]))
            o_k = k1[g] - k0[g]
            acc_ref[pl.ds(k1[g], h4), cols[g]] = (
                acc_ref[pl.ds(k1[g], h4), cols[g]]
                + recvs[g][0][pl.ds(o_k, h4), :]
            )

        rd2 = []
        for g in range(_NG):
            rd1[g].wait()
            acc_ref[pl.ds(k1[g], h4), cols[g]] = (
                acc_ref[pl.ds(k1[g], h4), cols[g]] + recvs[g][1][...]
            )
            rd2.append(start(2, g, par[g][2],
                             acc_ref.at[pl.ds(k1[g], h4), cols[g]], recvs[g][2]))

        ag1 = []
        for g in range(_NG):
            rd2[g].wait()
            acc_ref[pl.ds(k1[g], h4), cols[g]] = jnp.maximum(
                acc_ref[pl.ds(k1[g], h4), cols[g]] + recvs[g][2][...], 0
            )
            blk = acc_ref.at[pl.ds(k1[g], h4), cols[g]]
            ag1.append(start(3, g, par[g][1], blk, blk))

        for g in range(_NG):
            out_ref[pl.ds(k1[g], h4), cols[g]] = (
                acc_ref[pl.ds(k1[g], h4), cols[g]].astype(jnp.float32)
            )

        ag2 = []
        for g in range(_NG):
            ag1[g].wait()
            blka = acc_ref.at[pl.ds(k0[g], h4), cols[g]]
            blkb = acc_ref.at[pl.ds(k0[g] + h4, h4), cols[g]]
            ag2.append((start(4, g, par[g][0], blka, blka),
                        start(5, g, par[g][0], blkb, blkb)))
        for g in range(_NG):
            out_ref[pl.ds(s1[g], h4), cols[g]] = (
                acc_ref[pl.ds(s1[g], h4), cols[g]].astype(jnp.float32)
            )

        for g in range(_NG):
            ag2[g][0].wait()
            out_ref[pl.ds(s0[g], h4), cols[g]] = (
                acc_ref[pl.ds(s0[g], h4), cols[g]].astype(jnp.float32)
            )
        for g in range(_NG):
            ag2[g][1].wait()
            out_ref[pl.ds(s0[g] + h4, h4), cols[g]] = (
                acc_ref[pl.ds(s0[g] + h4, h4), cols[g]].astype(jnp.float32)
            )

    recv_shapes = []
    for _, w in _GROUPS:
        recv_shapes += [
            pltpu.VMEM((h2, w), jnp.bfloat16),
            pltpu.VMEM((h4, w), jnp.bfloat16),
            pltpu.VMEM((h4, w), jnp.bfloat16),
        ]

    return pl.pallas_call(
        body,
        out_shape=jax.ShapeDtypeStruct((m, n), jnp.float32),
        in_specs=[
            pl.BlockSpec(memory_space=pltpu.VMEM),
            pl.BlockSpec(memory_space=pltpu.VMEM),
        ],
        out_specs=pl.BlockSpec(memory_space=pltpu.VMEM),
        scratch_shapes=[
            pltpu.VMEM((m, n), jnp.bfloat16),
            *recv_shapes,
            pltpu.SemaphoreType.DMA((6 * _NG,)),
            pltpu.SemaphoreType.DMA((6 * _NG,)),
        ],
        compiler_params=(
            pltpu.CompilerParams(collective_id=0) if _COMM
            else pltpu.CompilerParams()
        ),
    )(x, w_mat)


# device time: 28378 ns/iter; 1.0001x vs baseline; 1.0001x over previous
import jax
import jax.numpy as jnp
from jax import lax
from jax.experimental import pallas as pl
from jax.experimental.pallas import tpu as pltpu

_GROUPS = [(0, 256), (256, 256), (512, 128), (640, 128), (768, 128), (896, 128)]
_NG = len(_GROUPS)
_COMM = True


def kernel(x, w_mat):
    m, _ = x.shape
    _, n = w_mat.shape
    h2, h4 = m // 2, m // 4

    def body(x_ref, w_ref, out_ref, *rest):
        accs = rest[:_NG]
        recvs = [rest[_NG + 3 * g: _NG + 3 * g + 3] for g in range(_NG)]
        send_sems, recv_sems = rest[4 * _NG], rest[4 * _NG + 1]

        i = lax.axis_index("i")
        q = i & 3
        ycrd = q >> 1
        xcrd = (q & 1) ^ ycrd
        zcrd = i >> 2
        py, px, pz = i ^ 3, i ^ 1, i ^ 4

        orders3 = [
            ((ycrd, xcrd, zcrd), (py, px, pz)),
            ((xcrd, zcrd, ycrd), (px, pz, py)),
            ((zcrd, ycrd, xcrd), (pz, py, px)),
        ]
        crd = [orders3[g % 3][0] for g in range(_NG)]
        par = [orders3[g % 3][1] for g in range(_NG)]
        cols = [pl.ds(off, w) for off, w in _GROUPS]

        k0 = [crd[g][0] * h2 for g in range(_NG)]
        k1 = [k0[g] + crd[g][1] * h4 for g in range(_NG)]
        s0 = [(1 - crd[g][0]) * h2 for g in range(_NG)]
        s1 = [k0[g] + (1 - crd[g][1]) * h4 for g in range(_NG)]

        if _COMM:
            barrier_sem = pltpu.get_barrier_semaphore()
            for nbr in (py, px, pz):
                pl.semaphore_signal(
                    barrier_sem, inc=1,
                    device_id=(nbr,), device_id_type=pl.DeviceIdType.MESH,
                )

        def mm(rows, g):
            return jnp.dot(
                x_ref[rows, :], w_ref[:, cols[g]],
                preferred_element_type=jnp.float32,
            ).astype(jnp.bfloat16)

        class _Noop:
            def wait(self):
                pass

        def start(r, g, partner, src, dst):
            if not _COMM:
                return _Noop()
            rdma = pltpu.make_async_remote_copy(
                src_ref=src, dst_ref=dst,
                send_sem=send_sems.at[6 * g + r], recv_sem=recv_sems.at[6 * g + r],
                device_id=(partner,), device_id_type=pl.DeviceIdType.MESH,
            )
            rdma.start()
            return rdma

        accs[0][pl.ds(s0[0], h2), :] = mm(pl.ds(s0[0], h2), 0)
        if _COMM:
            pl.semaphore_wait(barrier_sem, 3)
        rd = []
        for g in range(_NG):
            if g > 0:
                accs[g][pl.ds(s0[g], h2), :] = mm(pl.ds(s0[g], h2), g)
            rd.append(start(0, g, par[g][0],
                            accs[g].at[pl.ds(s0[g], h2)], recvs[g][0]))
        for g in range(_NG):
            accs[g][pl.ds(k0[g], h2), :] = mm(pl.ds(k0[g], h2), g)

        rd1 = []
        for g in range(_NG):
            rd[g].wait()
            o_s = s1[g] - k0[g]
            accs[g][pl.ds(s1[g], h4), :] = (
                accs[g][pl.ds(s1[g], h4), :] + recvs[g][0][pl.ds(o_s, h4), :]
            )
            rd1.append(start(1, g, par[g][1],
                             accs[g].at[pl.ds(s1[g], h4)], recvs[g][1]))
            o_k = k1[g] - k0[g]
            accs[g][pl.ds(k1[g], h4), :] = (
                accs[g][pl.ds(k1[g], h4), :] + recvs[g][0][pl.ds(o_k, h4), :]
            )

        rd2 = []
        for g in range(_NG):
            rd1[g].wait()
            accs[g][pl.ds(k1[g], h4), :] = (
                accs[g][pl.ds(k1[g], h4), :] + recvs[g][1][...]
            )
            rd2.append(start(2, g, par[g][2],
                             accs[g].at[pl.ds(k1[g], h4)], recvs[g][2]))

        ag1 = []
        for g in range(_NG):
            rd2[g].wait()
            accs[g][pl.ds(k1[g], h4), :] = jnp.maximum(
                accs[g][pl.ds(k1[g], h4), :] + recvs[g][2][...], 0
            )
            blk = accs[g].at[pl.ds(k1[g], h4)]
            ag1.append(start(3, g, par[g][1], blk, blk))

        for g in range(_NG):
            out_ref[pl.ds(k1[g], h4), cols[g]] = (
                accs[g][pl.ds(k1[g], h4), :].astype(jnp.float32)
            )

        ag2 = []
        for g in range(_NG):
            ag1[g].wait()
            blka = accs[g].at[pl.ds(k0[g], h4)]
            blkb = accs[g].at[pl.ds(k0[g] + h4, h4)]
            ag2.append((start(4, g, par[g][0], blka, blka),
                        start(5, g, par[g][0], blkb, blkb)))
        for g in range(_NG):
            out_ref[pl.ds(s1[g], h4), cols[g]] = (
                accs[g][pl.ds(s1[g], h4), :].astype(jnp.float32)
            )

        for g in range(_NG):
            ag2[g][0].wait()
            out_ref[pl.ds(s0[g], h4), cols[g]] = (
                accs[g][pl.ds(s0[g], h4), :].astype(jnp.float32)
            )
        for g in range(_NG):
            ag2[g][1].wait()
            out_ref[pl.ds(s0[g] + h4, h4), cols[g]] = (
                accs[g][pl.ds(s0[g] + h4, h4), :].astype(jnp.float32)
            )

    acc_shapes = [pltpu.VMEM((m, w), jnp.bfloat16) for _, w in _GROUPS]
    recv_shapes = []
    for _, w in _GROUPS:
        recv_shapes += [
            pltpu.VMEM((h2, w), jnp.bfloat16),
            pltpu.VMEM((h4, w), jnp.bfloat16),
            pltpu.VMEM((h4, w), jnp.bfloat16),
        ]

    return pl.pallas_call(
        body,
        out_shape=jax.ShapeDtypeStruct((m, n), jnp.float32),
        in_specs=[
            pl.BlockSpec(memory_space=pltpu.VMEM),
            pl.BlockSpec(memory_space=pltpu.VMEM),
        ],
        out_specs=pl.BlockSpec(memory_space=pltpu.VMEM),
        scratch_shapes=[
            *acc_shapes,
            *recv_shapes,
            pltpu.SemaphoreType.DMA((6 * _NG,)),
            pltpu.SemaphoreType.DMA((6 * _NG,)),
        ],
        compiler_params=(
            pltpu.CompilerParams(collective_id=0) if _COMM
            else pltpu.CompilerParams()
        ),
    )(x, w_mat)


# device time: 25833 ns/iter; 1.0986x vs baseline; 1.0985x over previous
import jax
import jax.numpy as jnp
from jax import lax
from jax.experimental import pallas as pl
from jax.experimental.pallas import tpu as pltpu

_GROUPS = [(128 * k, 128) for k in range(8)]
_NG = len(_GROUPS)
_NSEM = 7
_COMM = True
_VPU = True


def kernel(x, w_mat):
    m, _ = x.shape
    _, n = w_mat.shape
    h2, h4 = m // 2, m // 4

    def body(x_ref, w_ref, out_ref, *rest):
        accs = rest[:_NG]
        recvs = [rest[_NG + 3 * g: _NG + 3 * g + 3] for g in range(_NG)]
        send_sems, recv_sems = rest[4 * _NG], rest[4 * _NG + 1]

        i = lax.axis_index("i")
        q = i & 3
        ycrd = q >> 1
        xcrd = (q & 1) ^ ycrd
        zcrd = i >> 2
        py, px, pz = i ^ 3, i ^ 1, i ^ 4

        orders3 = [
            ((ycrd, xcrd, zcrd), (py, px, pz)),
            ((xcrd, zcrd, ycrd), (px, pz, py)),
            ((zcrd, ycrd, xcrd), (pz, py, px)),
        ]
        crd = [orders3[g % 3][0] for g in range(_NG)]
        par = [orders3[g % 3][1] for g in range(_NG)]
        cols = [pl.ds(off, w) for off, w in _GROUPS]

        k0 = [crd[g][0] * h2 for g in range(_NG)]
        k1 = [k0[g] + crd[g][1] * h4 for g in range(_NG)]
        s0 = [(1 - crd[g][0]) * h2 for g in range(_NG)]
        s1 = [k0[g] + (1 - crd[g][1]) * h4 for g in range(_NG)]
        o_s = [s1[g] - k0[g] for g in range(_NG)]
        o_k = [k1[g] - k0[g] for g in range(_NG)]
        d0 = [s0[g] + o_k[g] for g in range(_NG)]
        d1 = [s0[g] + o_s[g] for g in range(_NG)]

        if _COMM:
            barrier_sem = pltpu.get_barrier_semaphore()
            for nbr in (py, px, pz):
                pl.semaphore_signal(
                    barrier_sem, inc=1,
                    device_id=(nbr,), device_id_type=pl.DeviceIdType.MESH,
                )

        def mm(rows, g):
            return jnp.dot(
                x_ref[rows, :], w_ref[:, cols[g]],
                preferred_element_type=jnp.float32,
            ).astype(jnp.bfloat16)

        class _Noop:
            def wait(self):
                pass

        def start(r, g, partner, src, dst):
            if not _COMM:
                return _Noop()
            rdma = pltpu.make_async_remote_copy(
                src_ref=src, dst_ref=dst,
                send_sem=send_sems.at[_NSEM * g + r],
                recv_sem=recv_sems.at[_NSEM * g + r],
                device_id=(partner,), device_id_type=pl.DeviceIdType.MESH,
            )
            rdma.start()
            return rdma

        if _VPU:
            accs[0][pl.ds(s0[0], h2), :] = mm(pl.ds(s0[0], h2), 0)
        if _COMM:
            pl.semaphore_wait(barrier_sem, 3)
        rd0a, rd0b = [], []
        for g in range(_NG):
            if g > 0 and _VPU:
                accs[g][pl.ds(s0[g], h2), :] = mm(pl.ds(s0[g], h2), g)
            rd0a.append(start(0, g, par[g][0],
                              accs[g].at[pl.ds(s0[g] + o_s[g], h4)],
                              recvs[g][0].at[pl.ds(o_s[g], h4)]))
            rd0b.append(start(1, g, par[g][0],
                              accs[g].at[pl.ds(s0[g] + o_k[g], h4)],
                              recvs[g][0].at[pl.ds(o_k[g], h4)]))
        for g in range(_NG):
            if _VPU:
                accs[g][pl.ds(k0[g], h2), :] = mm(pl.ds(k0[g], h2), g)

        rd1 = []
        for g in range(_NG):
            rd0a[g].wait()
            if _VPU:
                accs[g][pl.ds(s1[g], h4), :] = (
                    accs[g][pl.ds(s1[g], h4), :]
                    + recvs[g][0][pl.ds(o_s[g], h4), :]
                )
            rd1.append(start(2, g, par[g][1],
                             accs[g].at[pl.ds(s1[g], h4)], recvs[g][1]))
            rd0b[g].wait()
            if _VPU:
                accs[g][pl.ds(k1[g], h4), :] = (
                    accs[g][pl.ds(k1[g], h4), :]
                    + recvs[g][0][pl.ds(o_k[g], h4), :]
                )

        rd2 = []
        for g in range(_NG):
            rd1[g].wait()
            if _VPU:
                accs[g][pl.ds(k1[g], h4), :] = (
                    accs[g][pl.ds(k1[g], h4), :] + recvs[g][1][...]
                )
            rd2.append(start(3, g, par[g][2],
                             accs[g].at[pl.ds(k1[g], h4)], recvs[g][2]))

        ag1, ag2a = [], []
        for g in range(_NG):
            rd2[g].wait()
            if _VPU:
                accs[g][pl.ds(k1[g], h4), :] = jnp.maximum(
                    accs[g][pl.ds(k1[g], h4), :] + recvs[g][2][...], 0
                )
            blk = accs[g].at[pl.ds(k1[g], h4)]
            ag1.append(start(4, g, par[g][1], blk, blk))
            ag2a.append(start(5, g, par[g][0], blk, blk))

        for g in range(_NG):
            if _VPU:
                out_ref[pl.ds(k1[g], h4), cols[g]] = (
                    accs[g][pl.ds(k1[g], h4), :].astype(jnp.float32)
                )

        ag2b = []
        for g in range(_NG):
            ag1[g].wait()
            blk = accs[g].at[pl.ds(s1[g], h4)]
            ag2b.append(start(6, g, par[g][0], blk, blk))
        for g in range(_NG):
            if _VPU:
                out_ref[pl.ds(s1[g], h4), cols[g]] = (
                    accs[g][pl.ds(s1[g], h4), :].astype(jnp.float32)
                )

        for g in range(_NG):
            ag2a[g].wait()
            if _VPU:
                out_ref[pl.ds(d0[g], h4), cols[g]] = (
                    accs[g][pl.ds(d0[g], h4), :].astype(jnp.float32)
                )
        for g in range(_NG):
            ag2b[g].wait()
            if _VPU:
                out_ref[pl.ds(d1[g], h4), cols[g]] = (
                    accs[g][pl.ds(d1[g], h4), :].astype(jnp.float32)
                )

    acc_shapes = [pltpu.VMEM((m, w), jnp.bfloat16) for _, w in _GROUPS]
    recv_shapes = []
    for _, w in _GROUPS:
        recv_shapes += [
            pltpu.VMEM((h2, w), jnp.bfloat16),
            pltpu.VMEM((h4, w), jnp.bfloat16),
            pltpu.VMEM((h4, w), jnp.bfloat16),
        ]

    return pl.pallas_call(
        body,
        out_shape=jax.ShapeDtypeStruct((m, n), jnp.float32),
        in_specs=[
            pl.BlockSpec(memory_space=pltpu.VMEM),
            pl.BlockSpec(memory_space=pltpu.VMEM),
        ],
        out_specs=pl.BlockSpec(memory_space=pltpu.VMEM),
        scratch_shapes=[
            *acc_shapes,
            *recv_shapes,
            pltpu.SemaphoreType.DMA((_NSEM * _NG,)),
            pltpu.SemaphoreType.DMA((_NSEM * _NG,)),
        ],
        compiler_params=(
            pltpu.CompilerParams(collective_id=0) if _COMM
            else pltpu.CompilerParams()
        ),
    )(x, w_mat)
